# Initial kernel scaffold; baseline (speedup 1.0000x reference)
import jax
import jax.numpy as jnp
from jax.experimental import pallas as pl


def kernel(z_e, codebook):
    raise NotImplementedError("write your pallas kernel here")


# speedup vs baseline: 1.3895x; 1.3895x over previous
import jax
import jax.numpy as jnp
from jax.experimental import pallas as pl
from jax.experimental.pallas import tpu as pltpu

_K = 1024
_D = 64
_BETA = 0.25
_BLK = 512


def _vq_body(z_ref, zsq_ref, cbt_ref, cb_ref, esq_ref,
             zq_ref, idx_ref, loss_ref, perp_ref,
             loss_acc, cnt_acc, *, n_rows, grid):
    i = pl.program_id(0)
    z = z_ref[...]
    dots = jax.lax.dot_general(
        z, cbt_ref[...], (((1,), (0,)), ((), ())),
        preferred_element_type=jnp.float32)
    d = (zsq_ref[...] + esq_ref[...]) - 2.0 * dots
    mind = jnp.min(d, axis=1, keepdims=True)
    iota = jax.lax.broadcasted_iota(jnp.int32, (_BLK, _K), 1)
    idx = jnp.min(jnp.where(d == mind, iota, _K),
                  axis=1, keepdims=True)
    idx_ref[...] = idx
    onehot = (iota == idx).astype(jnp.float32)
    zq = jax.lax.dot_general(
        onehot, cb_ref[...], (((1,), (0,)), ((), ())),
        preferred_element_type=jnp.float32)
    zq_ref[...] = zq

    @pl.when(i == 0)
    def _init():
        loss_acc[...] = jnp.zeros_like(loss_acc)
        cnt_acc[...] = jnp.zeros_like(cnt_acc)

    diff = z - zq
    loss_acc[...] += jnp.sum(diff * diff, axis=(0, 1), keepdims=True)
    cnt_acc[...] += jnp.sum(onehot, axis=0, keepdims=True)

    @pl.when(i == grid - 1)
    def _fini():
        loss_ref[...] = (1.0 + _BETA) * loss_acc[...] / (n_rows * _D)
        avg = cnt_acc[...] / n_rows
        ent = jnp.sum(avg * jnp.log(avg + 1e-12), axis=(0, 1), keepdims=True)
        perp_ref[...] = jnp.exp(-ent)


def kernel(z_e, codebook):
    z = z_e.reshape(-1, _D)
    n_rows = z.shape[0]
    grid = n_rows // _BLK
    zsq = jnp.sum(z ** 2, axis=1, keepdims=True)
    esq = jnp.sum(codebook ** 2, axis=1).reshape(1, _K)
    cbt = codebook.T

    body = lambda *refs: _vq_body(*refs, n_rows=n_rows, grid=grid)
    zq, idx, loss, perp = pl.pallas_call(
        body,
        grid=(grid,),
        in_specs=[
            pl.BlockSpec((_BLK, _D), lambda i: (i, 0)),
            pl.BlockSpec((_BLK, 1), lambda i: (i, 0)),
            pl.BlockSpec((_D, _K), lambda i: (0, 0)),
            pl.BlockSpec((_K, _D), lambda i: (0, 0)),
            pl.BlockSpec((1, _K), lambda i: (0, 0)),
        ],
        out_specs=[
            pl.BlockSpec((_BLK, _D), lambda i: (i, 0)),
            pl.BlockSpec((_BLK, 1), lambda i: (i, 0)),
            pl.BlockSpec((1, 1), lambda i: (0, 0)),
            pl.BlockSpec((1, 1), lambda i: (0, 0)),
        ],
        out_shape=[
            jax.ShapeDtypeStruct((n_rows, _D), jnp.float32),
            jax.ShapeDtypeStruct((n_rows, 1), jnp.int32),
            jax.ShapeDtypeStruct((1, 1), jnp.float32),
            jax.ShapeDtypeStruct((1, 1), jnp.float32),
        ],
        scratch_shapes=[
            pltpu.VMEM((1, 1), jnp.float32),
            pltpu.VMEM((1, _K), jnp.float32),
        ],
    )(z, zsq, cbt, codebook, esq)

    z_q = zq.reshape(z_e.shape)
    indices = idx.reshape(z_e.shape[:-1])
    return (z_q, indices, loss[0, 0], perp[0, 0])
